# baseline (speedup 1.0000x reference)
import functools

import jax
import jax.numpy as jnp
from jax import lax
from jax.experimental import pallas as pl
from jax.experimental.pallas import tpu as pltpu
from jax.experimental.pallas import tpu_sc as plsc

N = 10000
E = 320000
D_IN = 128
D_HID = 16
D_OUT = 128

NCORES = 2
NTILES = 16
S = 640
NP = NTILES * S
CH = 128
K = 160
K2 = K // NCORES
NB = 16
EP = NTILES * K * CH
G16 = S // 16

_SC_PARAMS = pltpu.CompilerParams(
    needs_layout_passes=False, use_tc_tiling_on_sc=False)


def _mesh():
    return plsc.VectorSubcoreMesh(
        core_axis_name="c", subcore_axis_name="s", num_cores=NCORES,
        num_subcores=NTILES)


def _rsqrt16(d):
    i = plsc.bitcast(d, jnp.int32)
    i = jnp.int32(0x5F3759DF) - lax.shift_right_logical(i, jnp.full((16,), 1, jnp.int32))
    y = plsc.bitcast(i, jnp.float32)
    for _ in range(4):
        y = y * (jnp.float32(1.5) - jnp.float32(0.5) * d * y * y)
    return y


def _bcast_lane(y, l):
    s = jnp.sum(jnp.where(lax.iota(jnp.int32, 16) == l, y, jnp.float32(0.0)))
    return jnp.full((16,), s, jnp.float32)


def _sweep(src_v, dst_v, rows_v, g_sh, acc_sh, sem, sem2):
    def esweep(jo, _):
        @pl.when(jo > 0)
        def _drain_prev():
            for b in range(NB):
                pltpu.make_async_copy(
                    rows_v.at[b], acc_sh.at[dst_v.at[0]], sem2).wait()
        gs = [pltpu.async_copy(g_sh.at[src_v.at[jo * NB + b]], rows_v.at[b], sem)
              for b in range(NB)]
        for b in range(NB):
            gs[b].wait()
            pltpu.async_copy(
                rows_v.at[b], acc_sh.at[dst_v.at[jo * NB + b]], sem2, add=True)
        return _
    lax.fori_loop(0, K2 // NB, esweep, None)
    for b in range(NB):
        pltpu.make_async_copy(rows_v.at[b], acc_sh.at[dst_v.at[0]], sem2).wait()


def _zero_acc_slice(rows_v, acc_sh, base):
    def zfill(r, _):
        for u in range(4):
            rows_v[0, r * 4 + u, :] = jnp.zeros((16,), jnp.float32)
        return _
    lax.fori_loop(0, CH // 4, zfill, None)
    for i in range(S // CH):
        pltpu.sync_copy(rows_v.at[0], acc_sh.at[pl.ds(base + i * CH, CH)])


def _deg_body(ei_hbm, degp_hbm, dst_v, hist_v, sum_v, degs_v, deg_sh, sem2):
    cid = lax.axis_index("c")
    sid = lax.axis_index("s")
    sl = pl.ds(sid * S, S)

    pltpu.sync_copy(ei_hbm.at[1, sid, pl.ds(cid * K2, K2)], dst_v)

    def zfill(i, _):
        for u in range(4):
            hist_v[pl.ds((i * 4 + u) * 16, 16)] = jnp.zeros((16,), jnp.float32)
        return _
    lax.fori_loop(0, NP // 64, zfill, None)

    one16 = jnp.full((16,), 1.0, jnp.float32)

    def count(j, _):
        for u in range(8):
            idx = dst_v[j, pl.ds(u * 16, 16)]
            plsc.addupdate_scatter(hist_v, [idx], one16)
        return _
    lax.fori_loop(0, K2, count, None)

    pltpu.sync_copy(hist_v, deg_sh.at[sid])
    plsc.subcore_barrier()
    pltpu.sync_copy(deg_sh.at[:, sl], sum_v)

    init = jnp.where(cid == 0, jnp.float32(1.0), jnp.float32(0.0))

    def sumtiles(i, _):
        acc = jnp.full((16,), init, jnp.float32)
        for t in range(NTILES):
            acc = acc + sum_v[t, pl.ds(i * 16, 16)]
        degs_v[pl.ds(i * 16, 16)] = acc
        return _
    lax.fori_loop(0, G16, sumtiles, None)
    pltpu.sync_copy(degs_v, degp_hbm.at[cid, sl])


@functools.cache
def _sc_deg():
    return pl.kernel(
        _deg_body,
        out_type=jax.ShapeDtypeStruct((NCORES, NP), jnp.float32),
        mesh=_mesh(),
        compiler_params=_SC_PARAMS,
        scratch_types=[
            pltpu.VMEM((K2, CH), jnp.int32),
            pltpu.VMEM((NP,), jnp.float32),
            pltpu.VMEM((NTILES, S), jnp.float32),
            pltpu.VMEM((S,), jnp.float32),
            pltpu.VMEM_SHARED((NTILES, NP), jnp.float32),
            pltpu.SemaphoreType.DMA,
        ],
    )


def _agg1_body(h1_hbm, degp_hbm, ei_hbm, s1p_hbm, dis_hbm,
               src_v, dst_v, degs_v, dis_v, disx_v, work_v, rows_v,
               g_sh, acc_sh, sem, sem2):
    cid = lax.axis_index("c")
    sid = lax.axis_index("s")
    base = sid * S
    sl = pl.ds(base, S)

    a_src = pltpu.async_copy(ei_hbm.at[0, sid, pl.ds(cid * K2, K2)], src_v, sem)
    a_dst = pltpu.async_copy(ei_hbm.at[1, sid, pl.ds(cid * K2, K2)], dst_v, sem)
    a_h1 = pltpu.async_copy(h1_hbm.at[sl], work_v, sem2)
    pltpu.sync_copy(degp_hbm.at[0, sl], degs_v)
    pltpu.sync_copy(degp_hbm.at[1, sl], dis_v)

    def mkdis(i, _):
        d = degs_v[pl.ds(i * 16, 16)] + dis_v[pl.ds(i * 16, 16)]
        dis_v[pl.ds(i * 16, 16)] = _rsqrt16(d)
        return _
    lax.fori_loop(0, G16, mkdis, None)

    def expdis(i, _):
        y = dis_v[pl.ds(i * 16, 16)]
        for l in range(16):
            disx_v[i * 16 + l, :] = _bcast_lane(y, l)
        return _
    lax.fori_loop(0, G16, expdis, None)

    a_h1.wait()

    def mkg1(i, _):
        for u in range(4):
            work_v[i * 4 + u, :] = work_v[i * 4 + u, :] * disx_v[i * 4 + u, :]
        return _
    lax.fori_loop(0, S // 4, mkg1, None)
    pltpu.sync_copy(work_v, g_sh.at[sl])

    @pl.when(cid == 0)
    def _seed():
        pltpu.sync_copy(work_v, acc_sh.at[sl])
        pltpu.sync_copy(disx_v, dis_hbm.at[sl])

    @pl.when(cid != 0)
    def _zero():
        _zero_acc_slice(rows_v, acc_sh, base)
    a_src.wait()
    a_dst.wait()
    plsc.subcore_barrier()

    _sweep(src_v, dst_v, rows_v, g_sh, acc_sh, sem, sem2)
    plsc.subcore_barrier()

    pltpu.sync_copy(acc_sh.at[sl], work_v)
    pltpu.sync_copy(work_v, s1p_hbm.at[cid, sl])


@functools.cache
def _sc_agg1():
    return pl.kernel(
        _agg1_body,
        out_type=(
            jax.ShapeDtypeStruct((NCORES, NP, D_HID), jnp.float32),
            jax.ShapeDtypeStruct((NP, D_HID), jnp.float32),
        ),
        mesh=_mesh(),
        compiler_params=_SC_PARAMS,
        scratch_types=[
            pltpu.VMEM((K2, CH), jnp.int32),
            pltpu.VMEM((K2, CH), jnp.int32),
            pltpu.VMEM((S,), jnp.float32),
            pltpu.VMEM((S,), jnp.float32),
            pltpu.VMEM((S, D_HID), jnp.float32),
            pltpu.VMEM((S, D_HID), jnp.float32),
            pltpu.VMEM((NB, CH, D_HID), jnp.float32),
            pltpu.VMEM_SHARED((NP, D_HID), jnp.float32),
            pltpu.VMEM_SHARED((NP, D_HID), jnp.float32),
            pltpu.SemaphoreType.DMA,
            pltpu.SemaphoreType.DMA,
        ],
    )


def _agg2_body(s1p_hbm, dis_hbm, b1_hbm, ei_hbm, s2p_hbm,
               src_v, dst_v, disx_v, work_v, tmp_v, b1_v, rows_v,
               g_sh, acc_sh, sem, sem2):
    cid = lax.axis_index("c")
    sid = lax.axis_index("s")
    base = sid * S
    sl = pl.ds(base, S)

    a_src = pltpu.async_copy(ei_hbm.at[0, sid, pl.ds(cid * K2, K2)], src_v, sem)
    a_dst = pltpu.async_copy(ei_hbm.at[1, sid, pl.ds(cid * K2, K2)], dst_v, sem)
    pltpu.sync_copy(dis_hbm.at[sl], disx_v)
    pltpu.sync_copy(s1p_hbm.at[0, sl], work_v)
    pltpu.sync_copy(s1p_hbm.at[1, sl], tmp_v)
    pltpu.sync_copy(b1_hbm, b1_v)
    b1r = b1_v[...]

    def mkg2(i, _):
        for u in range(4):
            d = disx_v[i * 4 + u, :]
            s = work_v[i * 4 + u, :] + tmp_v[i * 4 + u, :]
            r = jnp.maximum(d * s + b1r, jnp.float32(0.0))
            work_v[i * 4 + u, :] = r * d
        return _
    lax.fori_loop(0, S // 4, mkg2, None)
    pltpu.sync_copy(work_v, g_sh.at[sl])

    @pl.when(cid == 0)
    def _seed():
        pltpu.sync_copy(work_v, acc_sh.at[sl])

    @pl.when(cid != 0)
    def _zero():
        _zero_acc_slice(rows_v, acc_sh, base)
    a_src.wait()
    a_dst.wait()
    plsc.subcore_barrier()

    _sweep(src_v, dst_v, rows_v, g_sh, acc_sh, sem, sem2)
    plsc.subcore_barrier()

    pltpu.sync_copy(acc_sh.at[sl], work_v)
    pltpu.sync_copy(work_v, s2p_hbm.at[cid, sl])


@functools.cache
def _sc_agg2():
    return pl.kernel(
        _agg2_body,
        out_type=jax.ShapeDtypeStruct((NCORES, NP, D_HID), jnp.float32),
        mesh=_mesh(),
        compiler_params=_SC_PARAMS,
        scratch_types=[
            pltpu.VMEM((K2, CH), jnp.int32),
            pltpu.VMEM((K2, CH), jnp.int32),
            pltpu.VMEM((S, D_HID), jnp.float32),
            pltpu.VMEM((S, D_HID), jnp.float32),
            pltpu.VMEM((S, D_HID), jnp.float32),
            pltpu.VMEM((D_HID,), jnp.float32),
            pltpu.VMEM((NB, CH, D_HID), jnp.float32),
            pltpu.VMEM_SHARED((NP, D_HID), jnp.float32),
            pltpu.VMEM_SHARED((NP, D_HID), jnp.float32),
            pltpu.SemaphoreType.DMA,
            pltpu.SemaphoreType.DMA,
        ],
    )


def _mm1_body(x_ref, w_ref, o_ref):
    o_ref[pl.ds(0, N), :] = jnp.dot(
        x_ref[...], w_ref[...], preferred_element_type=jnp.float32)
    o_ref[pl.ds(N, NP - N), :] = jnp.zeros((NP - N, D_HID), jnp.float32)


_mm1 = pl.pallas_call(
    _mm1_body,
    out_shape=jax.ShapeDtypeStruct((NP, D_HID), jnp.float32),
)


def _out_body(s2p_ref, dis_ref, w_ref, b_ref, o_ref):
    a2 = dis_ref[...] * (s2p_ref[0, :, :] + s2p_ref[1, :, :])
    z = jnp.dot(a2[:N, :], w_ref[...],
                preferred_element_type=jnp.float32) + b_ref[...]
    m = jnp.max(z, axis=1, keepdims=True)
    z = z - m
    e = jnp.exp(z)
    s = jnp.sum(e, axis=1, keepdims=True)
    o_ref[...] = z - jnp.log(s)


_mmout = pl.pallas_call(
    _out_body,
    out_shape=jax.ShapeDtypeStruct((N, D_OUT), jnp.float32),
)


@jax.jit
def kernel(x, edge_index, W1, b1, W2, b2):
    h1p = _mm1(x, W1)

    eip = jnp.pad(edge_index, ((0, 0), (0, EP - E)),
                  constant_values=N).reshape(2, NTILES, K, CH)

    degp = _sc_deg()(eip)
    s1p, dis = _sc_agg1()(h1p, degp, eip)
    s2p = _sc_agg2()(s1p, dis, b1, eip)
    return _mmout(s2p, dis, W2, b2.reshape(1, D_OUT))


# speedup vs baseline: 1.0073x; 1.0073x over previous
import functools

import jax
import jax.numpy as jnp
from jax import lax
from jax.experimental import pallas as pl
from jax.experimental.pallas import tpu as pltpu
from jax.experimental.pallas import tpu_sc as plsc

N = 10000
E = 320000
D_IN = 128
D_HID = 16
D_OUT = 128

NCORES = 2
NTILES = 16
S = 640
NP = NTILES * S
CH = 128
K = 160
K2 = K // NCORES
NB = 16
EP = NTILES * K * CH
G16 = S // 16

_SC_PARAMS = pltpu.CompilerParams(
    needs_layout_passes=False, use_tc_tiling_on_sc=False)


def _mesh():
    return plsc.VectorSubcoreMesh(
        core_axis_name="c", subcore_axis_name="s", num_cores=NCORES,
        num_subcores=NTILES)


def _rsqrt16(d):
    i = plsc.bitcast(d, jnp.int32)
    i = jnp.int32(0x5F3759DF) - lax.shift_right_logical(i, jnp.full((16,), 1, jnp.int32))
    y = plsc.bitcast(i, jnp.float32)
    for _ in range(4):
        y = y * (jnp.float32(1.5) - jnp.float32(0.5) * d * y * y)
    return y


def _bcast_lane(y, l):
    s = jnp.sum(jnp.where(lax.iota(jnp.int32, 16) == l, y, jnp.float32(0.0)))
    return jnp.full((16,), s, jnp.float32)


def _sweep(src_v, dst_v, rows_v, g_sh, acc_sh, sem, sem2):
    def esweep(jo, _):
        @pl.when(jo > 0)
        def _drain_prev():
            for b in range(NB):
                pltpu.make_async_copy(
                    rows_v.at[b], acc_sh.at[dst_v.at[0]], sem2).wait()
        gs = [pltpu.async_copy(g_sh.at[src_v.at[jo * NB + b]], rows_v.at[b], sem)
              for b in range(NB)]
        for b in range(NB):
            gs[b].wait()
            pltpu.async_copy(
                rows_v.at[b], acc_sh.at[dst_v.at[jo * NB + b]], sem2, add=True)
        return _
    lax.fori_loop(0, K2 // NB, esweep, None)
    for b in range(NB):
        pltpu.make_async_copy(rows_v.at[b], acc_sh.at[dst_v.at[0]], sem2).wait()


def _zero_acc_slice(rows_v, acc_sh, base):
    def zfill(r, _):
        for u in range(4):
            rows_v[0, r * 4 + u, :] = jnp.zeros((16,), jnp.float32)
        return _
    lax.fori_loop(0, CH // 4, zfill, None)
    for i in range(S // CH):
        pltpu.sync_copy(rows_v.at[0], acc_sh.at[pl.ds(base + i * CH, CH)])


def _deg_body(ei_hbm, degp_hbm, dst_v, ones_v, degs_v, deg_sh, sem2):
    cid = lax.axis_index("c")
    sid = lax.axis_index("s")
    sl = pl.ds(sid * S, S)

    pltpu.sync_copy(ei_hbm.at[1, sid, pl.ds(cid * K2, K2)], dst_v)
    for i in range(8):
        ones_v[pl.ds(i * 16, 16)] = jnp.full((16,), 1.0, jnp.float32)

    init = jnp.where(cid == 0, jnp.float32(1.0), jnp.float32(0.0))

    def initdeg(i, _):
        degs_v[pl.ds(i * 16, 16)] = jnp.full((16,), init, jnp.float32)
        return _
    lax.fori_loop(0, G16, initdeg, None)
    pltpu.sync_copy(degs_v, deg_sh.at[sl])
    plsc.subcore_barrier()

    def degsweep(jo, _):
        for b in range(NB):
            pltpu.async_copy(ones_v, deg_sh.at[dst_v.at[jo * NB + b]], sem2,
                             add=True)
        return _
    lax.fori_loop(0, K2 // NB, degsweep, None)

    def degdrain(jo, _):
        for b in range(NB):
            pltpu.make_async_copy(ones_v, deg_sh.at[dst_v.at[0]], sem2).wait()
        return _
    lax.fori_loop(0, K2 // NB, degdrain, None)
    plsc.subcore_barrier()

    pltpu.sync_copy(deg_sh.at[sl], degs_v)
    pltpu.sync_copy(degs_v, degp_hbm.at[cid, sl])


@functools.cache
def _sc_deg():
    return pl.kernel(
        _deg_body,
        out_type=jax.ShapeDtypeStruct((NCORES, NP), jnp.float32),
        mesh=_mesh(),
        compiler_params=_SC_PARAMS,
        scratch_types=[
            pltpu.VMEM((K2, CH), jnp.int32),
            pltpu.VMEM((CH,), jnp.float32),
            pltpu.VMEM((S,), jnp.float32),
            pltpu.VMEM_SHARED((NP,), jnp.float32),
            pltpu.SemaphoreType.DMA,
        ],
    )


def _agg1_body(h1_hbm, degp_hbm, ei_hbm, s1p_hbm, dis_hbm,
               src_v, dst_v, degs_v, dis_v, disx_v, work_v, rows_v,
               g_sh, acc_sh, sem, sem2):
    cid = lax.axis_index("c")
    sid = lax.axis_index("s")
    base = sid * S
    sl = pl.ds(base, S)

    a_src = pltpu.async_copy(ei_hbm.at[0, sid, pl.ds(cid * K2, K2)], src_v, sem)
    a_dst = pltpu.async_copy(ei_hbm.at[1, sid, pl.ds(cid * K2, K2)], dst_v, sem)
    a_h1 = pltpu.async_copy(h1_hbm.at[sl], work_v, sem2)
    pltpu.sync_copy(degp_hbm.at[0, sl], degs_v)
    pltpu.sync_copy(degp_hbm.at[1, sl], dis_v)

    def mkdis(i, _):
        d = degs_v[pl.ds(i * 16, 16)] + dis_v[pl.ds(i * 16, 16)]
        dis_v[pl.ds(i * 16, 16)] = _rsqrt16(d)
        return _
    lax.fori_loop(0, G16, mkdis, None)

    def expdis(i, _):
        y = dis_v[pl.ds(i * 16, 16)]
        for l in range(16):
            disx_v[i * 16 + l, :] = _bcast_lane(y, l)
        return _
    lax.fori_loop(0, G16, expdis, None)

    a_h1.wait()

    def mkg1(i, _):
        for u in range(4):
            work_v[i * 4 + u, :] = work_v[i * 4 + u, :] * disx_v[i * 4 + u, :]
        return _
    lax.fori_loop(0, S // 4, mkg1, None)
    pltpu.sync_copy(work_v, g_sh.at[sl])

    @pl.when(cid == 0)
    def _seed():
        pltpu.sync_copy(work_v, acc_sh.at[sl])
        pltpu.sync_copy(disx_v, dis_hbm.at[sl])

    @pl.when(cid != 0)
    def _zero():
        _zero_acc_slice(rows_v, acc_sh, base)
    a_src.wait()
    a_dst.wait()
    plsc.subcore_barrier()

    _sweep(src_v, dst_v, rows_v, g_sh, acc_sh, sem, sem2)
    plsc.subcore_barrier()

    pltpu.sync_copy(acc_sh.at[sl], work_v)
    pltpu.sync_copy(work_v, s1p_hbm.at[cid, sl])


@functools.cache
def _sc_agg1():
    return pl.kernel(
        _agg1_body,
        out_type=(
            jax.ShapeDtypeStruct((NCORES, NP, D_HID), jnp.float32),
            jax.ShapeDtypeStruct((NP, D_HID), jnp.float32),
        ),
        mesh=_mesh(),
        compiler_params=_SC_PARAMS,
        scratch_types=[
            pltpu.VMEM((K2, CH), jnp.int32),
            pltpu.VMEM((K2, CH), jnp.int32),
            pltpu.VMEM((S,), jnp.float32),
            pltpu.VMEM((S,), jnp.float32),
            pltpu.VMEM((S, D_HID), jnp.float32),
            pltpu.VMEM((S, D_HID), jnp.float32),
            pltpu.VMEM((NB, CH, D_HID), jnp.float32),
            pltpu.VMEM_SHARED((NP, D_HID), jnp.float32),
            pltpu.VMEM_SHARED((NP, D_HID), jnp.float32),
            pltpu.SemaphoreType.DMA,
            pltpu.SemaphoreType.DMA,
        ],
    )


def _agg2_body(s1p_hbm, dis_hbm, b1_hbm, ei_hbm, s2p_hbm,
               src_v, dst_v, disx_v, work_v, tmp_v, b1_v, rows_v,
               g_sh, acc_sh, sem, sem2):
    cid = lax.axis_index("c")
    sid = lax.axis_index("s")
    base = sid * S
    sl = pl.ds(base, S)

    a_src = pltpu.async_copy(ei_hbm.at[0, sid, pl.ds(cid * K2, K2)], src_v, sem)
    a_dst = pltpu.async_copy(ei_hbm.at[1, sid, pl.ds(cid * K2, K2)], dst_v, sem)
    pltpu.sync_copy(dis_hbm.at[sl], disx_v)
    pltpu.sync_copy(s1p_hbm.at[0, sl], work_v)
    pltpu.sync_copy(s1p_hbm.at[1, sl], tmp_v)
    pltpu.sync_copy(b1_hbm, b1_v)
    b1r = b1_v[...]

    def mkg2(i, _):
        for u in range(4):
            d = disx_v[i * 4 + u, :]
            s = work_v[i * 4 + u, :] + tmp_v[i * 4 + u, :]
            r = jnp.maximum(d * s + b1r, jnp.float32(0.0))
            work_v[i * 4 + u, :] = r * d
        return _
    lax.fori_loop(0, S // 4, mkg2, None)
    pltpu.sync_copy(work_v, g_sh.at[sl])

    @pl.when(cid == 0)
    def _seed():
        pltpu.sync_copy(work_v, acc_sh.at[sl])

    @pl.when(cid != 0)
    def _zero():
        _zero_acc_slice(rows_v, acc_sh, base)
    a_src.wait()
    a_dst.wait()
    plsc.subcore_barrier()

    _sweep(src_v, dst_v, rows_v, g_sh, acc_sh, sem, sem2)
    plsc.subcore_barrier()

    pltpu.sync_copy(acc_sh.at[sl], work_v)
    pltpu.sync_copy(work_v, s2p_hbm.at[cid, sl])


@functools.cache
def _sc_agg2():
    return pl.kernel(
        _agg2_body,
        out_type=jax.ShapeDtypeStruct((NCORES, NP, D_HID), jnp.float32),
        mesh=_mesh(),
        compiler_params=_SC_PARAMS,
        scratch_types=[
            pltpu.VMEM((K2, CH), jnp.int32),
            pltpu.VMEM((K2, CH), jnp.int32),
            pltpu.VMEM((S, D_HID), jnp.float32),
            pltpu.VMEM((S, D_HID), jnp.float32),
            pltpu.VMEM((S, D_HID), jnp.float32),
            pltpu.VMEM((D_HID,), jnp.float32),
            pltpu.VMEM((NB, CH, D_HID), jnp.float32),
            pltpu.VMEM_SHARED((NP, D_HID), jnp.float32),
            pltpu.VMEM_SHARED((NP, D_HID), jnp.float32),
            pltpu.SemaphoreType.DMA,
            pltpu.SemaphoreType.DMA,
        ],
    )


def _mm1_body(x_ref, w_ref, o_ref):
    o_ref[pl.ds(0, N), :] = jnp.dot(
        x_ref[...], w_ref[...], preferred_element_type=jnp.float32)
    o_ref[pl.ds(N, NP - N), :] = jnp.zeros((NP - N, D_HID), jnp.float32)


_mm1 = pl.pallas_call(
    _mm1_body,
    out_shape=jax.ShapeDtypeStruct((NP, D_HID), jnp.float32),
)


def _out_body(s2p_ref, dis_ref, w_ref, b_ref, o_ref):
    a2 = dis_ref[...] * (s2p_ref[0, :, :] + s2p_ref[1, :, :])
    z = jnp.dot(a2[:N, :], w_ref[...],
                preferred_element_type=jnp.float32) + b_ref[...]
    m = jnp.max(z, axis=1, keepdims=True)
    z = z - m
    e = jnp.exp(z)
    s = jnp.sum(e, axis=1, keepdims=True)
    o_ref[...] = z - jnp.log(s)


_mmout = pl.pallas_call(
    _out_body,
    out_shape=jax.ShapeDtypeStruct((N, D_OUT), jnp.float32),
)


@jax.jit
def kernel(x, edge_index, W1, b1, W2, b2):
    h1p = _mm1(x, W1)

    eip = jnp.pad(edge_index, ((0, 0), (0, EP - E)),
                  constant_values=N).reshape(2, NTILES, K, CH)

    degp = _sc_deg()(eip)
    s1p, dis = _sc_agg1()(h1p, degp, eip)
    s2p = _sc_agg2()(s1p, dis, b1, eip)
    return _mmout(s2p, dis, W2, b2.reshape(1, D_OUT))
